# baseline (speedup 1.0000x reference)
import functools

import jax
import jax.numpy as jnp
from jax import lax
from jax.experimental import pallas as pl
from jax.experimental.pallas import tpu as pltpu
from jax.experimental.pallas import tpu_sc as plsc

NUM_CLASSES = 100000
EMBED_DIM = 128
BATCH = 16384
FIELDS = 100
FPAD = 104

NC = 2
NS = 16
NW = NC * NS

B_PER_W = BATCH // NW
NBUF = 4
N_ROUNDS = B_PER_W // NBUF


def _make_sc_gather():
    mesh = plsc.VectorSubcoreMesh(core_axis_name="c", subcore_axis_name="s")

    @functools.partial(
        pl.kernel,
        mesh=mesh,
        out_type=jax.ShapeDtypeStruct((BATCH, FIELDS, EMBED_DIM),
                                      jnp.float32),
        scratch_types=[
            pltpu.VMEM((B_PER_W * FPAD,), jnp.int32),
            pltpu.VMEM((NBUF, FIELDS, EMBED_DIM), jnp.float32),
            pltpu.SemaphoreType.DMA((NBUF,)),
            pltpu.SemaphoreType.DMA((NBUF,)),
        ],
        compiler_params=pltpu.CompilerParams(use_tc_tiling_on_sc=True),
    )
    def sc_gather(lab_hbm, tab_hbm, out_hbm, idx_v, rows_v, gsem, osem):
        wid = lax.axis_index("s") * NC + lax.axis_index("c")
        base = wid * B_PER_W

        pltpu.sync_copy(lab_hbm.at[pl.ds(base * FPAD, B_PER_W * FPAD)],
                        idx_v)

        for b in range(NBUF):
            pltpu.async_copy(tab_hbm.at[idx_v.at[pl.ds(b * FPAD, FIELDS)]],
                             rows_v.at[b], gsem.at[b])

        def round_body(r, _):
            for b in range(NBUF):
                c = r * NBUF + b
                pltpu.make_async_copy(
                    tab_hbm.at[idx_v.at[pl.ds(c * FPAD, FIELDS)]],
                    rows_v.at[b], gsem.at[b]).wait()
                pltpu.async_copy(rows_v.at[b], out_hbm.at[base + c],
                                 osem.at[b])

                @pl.when(r < N_ROUNDS - 1)
                def _():
                    pltpu.make_async_copy(rows_v.at[b],
                                          out_hbm.at[base + c],
                                          osem.at[b]).wait()
                    pltpu.async_copy(
                        tab_hbm.at[idx_v.at[pl.ds((c + NBUF) * FPAD,
                                                  FIELDS)]],
                        rows_v.at[b], gsem.at[b])
            return 0

        lax.fori_loop(0, N_ROUNDS, round_body, 0)

        for b in range(NBUF):
            c = (N_ROUNDS - 1) * NBUF + b
            pltpu.make_async_copy(rows_v.at[b], out_hbm.at[base + c],
                                  osem.at[b]).wait()

    return sc_gather


_sc_gather = _make_sc_gather()


def kernel(labels, table):
    labels_r = labels.astype(jnp.int32).reshape(NW, BATCH // NW, FIELDS)
    labels_p = jnp.pad(labels_r, ((0, 0), (0, 0), (0, FPAD - FIELDS)))
    return _sc_gather(labels_p.reshape(-1), table)


# speedup vs baseline: 1.8890x; 1.8890x over previous
import functools

import jax
import jax.numpy as jnp
from jax import lax
from jax.experimental import pallas as pl
from jax.experimental.pallas import tpu as pltpu
from jax.experimental.pallas import tpu_sc as plsc

NUM_CLASSES = 100000
EMBED_DIM = 128
BATCH = 16384
FIELDS = 100

NC = 2
NS = 16
NW = NC * NS

NUM_ROWS = BATCH * FIELDS
ROWS_PER_W = NUM_ROWS // NW
CHUNK = 128
N_CHUNKS = ROWS_PER_W // CHUNK
NBUF = 4
N_ROUNDS = N_CHUNKS // NBUF


def _make_sc_gather():
    mesh = plsc.VectorSubcoreMesh(core_axis_name="c", subcore_axis_name="s")

    @functools.partial(
        pl.kernel,
        mesh=mesh,
        out_type=jax.ShapeDtypeStruct((NUM_ROWS, EMBED_DIM), jnp.float32),
        scratch_types=[
            pltpu.VMEM((N_CHUNKS, CHUNK), jnp.int32),
            pltpu.VMEM((NBUF, CHUNK, EMBED_DIM), jnp.float32),
            pltpu.SemaphoreType.DMA((NBUF,)),
            pltpu.SemaphoreType.DMA((NBUF,)),
        ],
        compiler_params=pltpu.CompilerParams(use_tc_tiling_on_sc=True),
    )
    def sc_gather(lab_hbm, tab_hbm, out_hbm, idx_v, rows_v, gsem, osem):
        wid = lax.axis_index("s") * NC + lax.axis_index("c")
        base = wid * ROWS_PER_W

        pltpu.sync_copy(lab_hbm.at[wid], idx_v)

        for b in range(NBUF):
            pltpu.async_copy(tab_hbm.at[idx_v.at[b]], rows_v.at[b],
                             gsem.at[b])

        def round_body(r, _):
            for b in range(NBUF):
                c = r * NBUF + b
                pltpu.make_async_copy(tab_hbm.at[idx_v.at[c]], rows_v.at[b],
                                      gsem.at[b]).wait()
                pltpu.async_copy(rows_v.at[b],
                                 out_hbm.at[pl.ds(base + c * CHUNK, CHUNK)],
                                 osem.at[b])

                @pl.when(r < N_ROUNDS - 1)
                def _():
                    pltpu.make_async_copy(
                        rows_v.at[b],
                        out_hbm.at[pl.ds(base + c * CHUNK, CHUNK)],
                        osem.at[b]).wait()
                    pltpu.async_copy(tab_hbm.at[idx_v.at[c + NBUF]],
                                     rows_v.at[b], gsem.at[b])
            return 0

        lax.fori_loop(0, N_ROUNDS, round_body, 0)

        for b in range(NBUF):
            c = (N_ROUNDS - 1) * NBUF + b
            pltpu.make_async_copy(rows_v.at[b],
                                  out_hbm.at[pl.ds(base + c * CHUNK, CHUNK)],
                                  osem.at[b]).wait()

    return sc_gather


_sc_gather = _make_sc_gather()


def kernel(labels, table):
    ids = labels.astype(jnp.int32).T.reshape(NW, N_CHUNKS, CHUNK)
    out = _sc_gather(ids, table)
    return out.reshape(FIELDS, BATCH, EMBED_DIM).transpose(1, 0, 2)
